# baseline (speedup 1.0000x reference)
import jax
import jax.numpy as jnp
from jax import lax
from jax.experimental import pallas as pl
from jax.experimental.pallas import tpu as pltpu
from jax.experimental.pallas import tpu_sc as plsc

NUM_EMBEDDINGS = 1000000
EMBEDDING_DIM = 64
BATCH = 16384
N_FIELDS = 26

NC = 2
NS = 16
NW = NC * NS
LANES = 16

B_TOTAL = BATCH * N_FIELDS
CHUNK = 128
N_BLOCKS = B_TOTAL // CHUNK
BLK_PER_W = N_BLOCKS // NW
NBUF = 4
K = 3
NOBUF = 2
BBLKS = BATCH // CHUNK


def _body(ids_hbm, table_hbm, out_hbm, idx_v, rows_v, tr_v, in_sems, out_sems):
    wid = lax.axis_index("s") * NC + lax.axis_index("c")
    g0 = wid * BLK_PER_W

    f0 = jnp.minimum(g0 // BBLKS, N_FIELDS - 2)
    pltpu.sync_copy(ids_hbm.at[pl.ds(f0, 2)], idx_v)

    def idx_slice(i):
        g = g0 + i
        fl = g // BBLKS - f0
        b0 = (g % BBLKS) * CHUNK
        return idx_v.at[fl, pl.ds(b0, CHUNK)]

    def issue_gather(i, buf):
        pltpu.async_copy(table_hbm.at[idx_slice(i)], rows_v.at[buf],
                         in_sems.at[buf])

    def wait_gather(i, buf):
        pltpu.make_async_copy(table_hbm.at[idx_slice(i)], rows_v.at[buf],
                              in_sems.at[buf]).wait()

    def out_slice(i, obuf):
        g = g0 + i
        f = g // BBLKS
        b0 = (g % BBLKS) * CHUNK
        return tr_v.at[obuf], out_hbm.at[f, :, pl.ds(b0, CHUNK)]

    def issue_store(i, obuf):
        src, dst = out_slice(i, obuf)
        pltpu.async_copy(src, dst, out_sems.at[obuf])

    def wait_store(i, obuf):
        src, dst = out_slice(i, obuf)
        pltpu.make_async_copy(src, dst, out_sems.at[obuf]).wait()

    def transpose_block(buf, obuf):
        rows = rows_v.at[buf]
        tr = tr_v.at[obuf]

        iv = lax.iota(jnp.int32, LANES)

        def d_body(dt, _):
            d0 = dt * LANES
            for t in range(LANES):
                dvb = ((iv + t) & (LANES - 1)) + d0
                xs = []
                for j0 in range(CHUNK // LANES):
                    jv = iv + (j0 * LANES)
                    xs.append(plsc.load_gather(rows, [jv, dvb]))
                for j0 in range(CHUNK // LANES):
                    jv = iv + (j0 * LANES)
                    plsc.store_scatter(tr, [dvb, jv], xs[j0])
            return 0

        lax.fori_loop(0, EMBEDDING_DIM // LANES, d_body, 0, unroll=False)

    def step(i, bi, *, head, tail):
        obuf = bi % NOBUF
        if not tail:
            issue_gather(i + K, (bi + K) % NBUF)
        wait_gather(i, bi)
        if not head:
            wait_store(i - NOBUF, obuf)
        transpose_block(bi, obuf)
        issue_store(i, obuf)

    for i in range(K):
        issue_gather(i, i)
    for bi in range(NBUF):
        step(bi, bi, head=(bi < NOBUF), tail=False)

    def loop_body(g, _):
        i0 = g * NBUF
        for bi in range(NBUF):
            step(i0 + bi, bi, head=False, tail=False)
        return 0

    lax.fori_loop(1, BLK_PER_W // NBUF - 1, loop_body, 0, unroll=False)

    i0 = BLK_PER_W - NBUF
    for bi in range(NBUF):
        step(i0 + bi, bi, head=False, tail=(bi + K >= NBUF))

    for i in range(BLK_PER_W - NOBUF, BLK_PER_W):
        wait_store(i, i % NOBUF)


@jax.jit
def _gather(ids_grouped, table):
    mesh = plsc.VectorSubcoreMesh(core_axis_name="c", subcore_axis_name="s",
                                  num_cores=NC, num_subcores=NS)
    f = pl.kernel(
        _body,
        out_type=jax.ShapeDtypeStruct((N_FIELDS, EMBEDDING_DIM, BATCH),
                                      jnp.float32),
        mesh=mesh,
        scratch_types=[
            pltpu.VMEM((2, BATCH), jnp.int32),
            pltpu.VMEM((NBUF, CHUNK, EMBEDDING_DIM), jnp.float32),
            pltpu.VMEM((NOBUF, EMBEDDING_DIM, CHUNK), jnp.float32),
            pltpu.SemaphoreType.DMA((NBUF,)),
            pltpu.SemaphoreType.DMA((NOBUF,)),
        ],
        compiler_params=pltpu.CompilerParams(use_tc_tiling_on_sc=False,
                                             needs_layout_passes=False,
                                             disable_bounds_checks=True),
    )
    return f(ids_grouped, table)


def kernel(ids, table):
    out_phys = _gather(ids.T.astype(jnp.int32), table)
    return out_phys.transpose(2, 0, 1)


# speedup vs baseline: 1.4359x; 1.4359x over previous
import jax
import jax.numpy as jnp
from jax import lax
from jax.experimental import pallas as pl
from jax.experimental.pallas import tpu as pltpu
from jax.experimental.pallas import tpu_sc as plsc

NUM_EMBEDDINGS = 1000000
EMBEDDING_DIM = 64
BATCH = 16384
N_FIELDS = 26

NC = 2
NS = 16
NW = NC * NS
LANES = 16

B_TOTAL = BATCH * N_FIELDS
CHUNK = 128
N_BLOCKS = B_TOTAL // CHUNK
BLK_PER_W = N_BLOCKS // NW
NBUF = 4
K = 3
NOBUF = 2
BBLKS = BATCH // CHUNK

PADDED = 128
N_TCOL = (NUM_EMBEDDINGS + CHUNK - 1) // CHUNK
TCOL_MAIN = 7808
TCOL_PER_W = TCOL_MAIN // NW


def _diag_transpose(src, dst, n_d, n_j, iv):
    def d_body(dt, _):
        d0 = dt * LANES
        for t in range(LANES):
            dvb = ((iv + t) & (LANES - 1)) + d0
            xs = []
            for j0 in range(n_j // LANES):
                jv = iv + (j0 * LANES)
                xs.append(plsc.load_gather(src, [jv, dvb]))
            for j0 in range(n_j // LANES):
                jv = iv + (j0 * LANES)
                plsc.store_scatter(dst, [dvb, jv], xs[j0])
        return 0

    lax.fori_loop(0, n_d // LANES, d_body, 0, unroll=False)


def _detile_body(tt_hbm, out_hbm, slab_v, ob_v, in_sems, out_sems):
    wid = lax.axis_index("s") * NC + lax.axis_index("c")
    iv = lax.iota(jnp.int32, LANES)

    def tc_of(i):
        return i * NW + wid

    def issue_read(i, buf):
        pltpu.async_copy(tt_hbm.at[:, pl.ds(tc_of(i) * CHUNK, CHUNK)],
                         slab_v.at[buf], in_sems.at[buf])

    def wait_read(i, buf):
        pltpu.make_async_copy(tt_hbm.at[:, pl.ds(tc_of(i) * CHUNK, CHUNK)],
                              slab_v.at[buf], in_sems.at[buf]).wait()

    def issue_store(i, obuf):
        pltpu.async_copy(ob_v.at[obuf],
                         out_hbm.at[pl.ds(tc_of(i) * CHUNK, CHUNK)],
                         out_sems.at[obuf])

    def wait_store(obuf):
        pltpu.make_async_copy(ob_v.at[obuf],
                              out_hbm.at[pl.ds(0, CHUNK)],
                              out_sems.at[obuf]).wait()

    def step(i, bi, *, head, tail):
        obuf = bi % NOBUF
        if not tail:
            issue_read(i + K, (bi + K) % NBUF)
        wait_read(i, bi)
        if not head:
            wait_store(obuf)
        _diag_transpose(slab_v.at[bi], ob_v.at[obuf],
                        EMBEDDING_DIM, CHUNK, iv)
        issue_store(i, obuf)

    for i in range(K):
        issue_read(i, i)
    for bi in range(NBUF):
        step(bi, bi, head=(bi < NOBUF), tail=False)

    def loop_body(g, _):
        i0 = g * NBUF
        for bi in range(NBUF):
            step(i0 + bi, bi, head=False, tail=False)
        return 0

    lax.fori_loop(1, TCOL_PER_W // NBUF - 1, loop_body, 0, unroll=False)

    i0 = TCOL_PER_W - NBUF
    for bi in range(NBUF):
        step(i0 + bi, bi, head=False, tail=(bi + K >= NBUF))
    for bi in range(NOBUF):
        wait_store(bi)

    @pl.when(wid >= NW - 4)
    def _full_tail():
        tc = TCOL_MAIN + (wid - (NW - 4))
        pltpu.sync_copy(tt_hbm.at[:, pl.ds(tc * CHUNK, CHUNK)], slab_v.at[0])
        _diag_transpose(slab_v.at[0], ob_v.at[0], EMBEDDING_DIM, CHUNK, iv)
        pltpu.sync_copy(ob_v.at[0], out_hbm.at[pl.ds(tc * CHUNK, CHUNK)])


T0 = TCOL_MAIN * CHUNK + 4 * CHUNK


def _gather_body(ids_hbm, table_hbm, tail_hbm, out_hbm,
                 idx_v, rows_v, tr_v, tail_v, in_sems, out_sems):
    wid = lax.axis_index("s") * NC + lax.axis_index("c")
    g0 = wid * BLK_PER_W
    iv = lax.iota(jnp.int32, LANES)

    f0 = jnp.minimum(g0 // BBLKS, N_FIELDS - 2)
    pltpu.sync_copy(ids_hbm.at[pl.ds(f0, 2)], idx_v)
    pltpu.sync_copy(tail_hbm, tail_v)

    def patch_tail(i, obuf):
        g = g0 + i
        fl = g // BBLKS - f0
        b0 = (g % BBLKS) * CHUNK
        tr = tr_v.at[obuf]

        def cnt_body(j0, c):
            idv = idx_v[fl, pl.ds(b0 + j0 * LANES, LANES)]
            return c + plsc.all_reduce_population_count(idv >= T0)[0]

        cnt = lax.fori_loop(0, CHUNK // LANES, cnt_body, jnp.int32(0),
                            unroll=False)

        @pl.when(cnt > 0)
        def _fix():
            def fix_j(j0, _):
                idv = idx_v[fl, pl.ds(b0 + j0 * LANES, LANES)]
                m = idv >= T0
                jv = iv + (j0 * LANES)
                rowv = jnp.where(m, idv - T0, 0)

                def fix_d(d, __):
                    dv = jnp.full((LANES,), 0, jnp.int32) + d
                    v = plsc.load_gather(tail_v, [rowv, dv], mask=m)
                    plsc.store_scatter(tr, [dv, jv], v, mask=m)
                    return 0

                return lax.fori_loop(0, EMBEDDING_DIM, fix_d, 0,
                                     unroll=False)

            lax.fori_loop(0, CHUNK // LANES, fix_j, 0, unroll=False)

    def idx_slice(i):
        g = g0 + i
        fl = g // BBLKS - f0
        b0 = (g % BBLKS) * CHUNK
        return idx_v.at[fl, pl.ds(b0, CHUNK)]

    def issue_gather(i, buf):
        pltpu.async_copy(table_hbm.at[idx_slice(i)], rows_v.at[buf],
                         in_sems.at[buf])

    def wait_gather(i, buf):
        pltpu.make_async_copy(table_hbm.at[idx_slice(i)], rows_v.at[buf],
                              in_sems.at[buf]).wait()

    def out_slice(i, obuf):
        g = g0 + i
        f = g // BBLKS
        b0 = (g % BBLKS) * CHUNK
        return tr_v.at[obuf], out_hbm.at[f, :, pl.ds(b0, CHUNK)]

    def issue_store(i, obuf):
        src, dst = out_slice(i, obuf)
        pltpu.async_copy(src, dst, out_sems.at[obuf])

    def wait_store(i, obuf):
        src, dst = out_slice(i, obuf)
        pltpu.make_async_copy(src, dst, out_sems.at[obuf]).wait()

    def step(i, bi, *, head, tail):
        obuf = bi % NOBUF
        if not tail:
            issue_gather(i + K, (bi + K) % NBUF)
        wait_gather(i, bi)
        if not head:
            wait_store(i - NOBUF, obuf)
        _diag_transpose(rows_v.at[bi], tr_v.at[obuf],
                        EMBEDDING_DIM, CHUNK, iv)
        patch_tail(i, obuf)
        issue_store(i, obuf)

    for i in range(K):
        issue_gather(i, i)
    for bi in range(NBUF):
        step(bi, bi, head=(bi < NOBUF), tail=False)

    def loop_body(g, _):
        i0 = g * NBUF
        for bi in range(NBUF):
            step(i0 + bi, bi, head=False, tail=False)
        return 0

    lax.fori_loop(1, BLK_PER_W // NBUF - 1, loop_body, 0, unroll=False)

    i0 = BLK_PER_W - NBUF
    for bi in range(NBUF):
        step(i0 + bi, bi, head=False, tail=(bi + K >= NBUF))

    for i in range(BLK_PER_W - NOBUF, BLK_PER_W):
        wait_store(i, i % NOBUF)


@jax.jit
def _gather(ids_t, table):
    mesh = plsc.VectorSubcoreMesh(core_axis_name="c", subcore_axis_name="s",
                                  num_cores=NC, num_subcores=NS)
    detile = pl.kernel(
        _detile_body,
        out_type=jax.ShapeDtypeStruct((NUM_EMBEDDINGS, PADDED), jnp.float32),
        mesh=mesh,
        scratch_types=[
            pltpu.VMEM((NBUF, EMBEDDING_DIM, CHUNK), jnp.float32),
            pltpu.VMEM((NOBUF, CHUNK, PADDED), jnp.float32),
            pltpu.SemaphoreType.DMA((NBUF,)),
            pltpu.SemaphoreType.DMA((NOBUF,)),
        ],
        compiler_params=pltpu.CompilerParams(use_tc_tiling_on_sc=True,
                                             needs_layout_passes=False,
                                             disable_bounds_checks=True),
    )
    table_rm = detile(table.T)

    gather = pl.kernel(
        _gather_body,
        out_type=jax.ShapeDtypeStruct((N_FIELDS, EMBEDDING_DIM, BATCH),
                                      jnp.float32),
        mesh=mesh,
        scratch_types=[
            pltpu.VMEM((2, BATCH), jnp.int32),
            pltpu.VMEM((NBUF, CHUNK, PADDED), jnp.float32),
            pltpu.VMEM((NOBUF, EMBEDDING_DIM, CHUNK), jnp.float32),
            pltpu.VMEM((NUM_EMBEDDINGS - T0, PADDED), jnp.float32),
            pltpu.SemaphoreType.DMA((NBUF,)),
            pltpu.SemaphoreType.DMA((NOBUF,)),
        ],
        compiler_params=pltpu.CompilerParams(use_tc_tiling_on_sc=False,
                                             needs_layout_passes=False,
                                             disable_bounds_checks=True),
    )
    tail_pad = jnp.pad(lax.slice(table, (T0, 0), (NUM_EMBEDDINGS, EMBEDDING_DIM)),
                       ((0, 0), (0, PADDED - EMBEDDING_DIM)))
    return gather(ids_t, table_rm, tail_pad)


def kernel(ids, table):
    out_phys = _gather(ids.T.astype(jnp.int32), table)
    return out_phys.transpose(2, 0, 1)
